# baseline (speedup 1.0000x reference)
import functools
import math

import numpy as np
import jax
import jax.numpy as jnp
from jax import lax
from jax.experimental import pallas as pl
from jax.experimental.pallas import tpu as pltpu
from jax.experimental.pallas import tpu_sc as plsc

B, N, E = 2, 8192, 768
H = 12
Dh = E // H
P = 64
BUCKET_SIZES = (256, 64, 16)
NL = len(BUCKET_SIZES)
BH = B * H
PK = 4 * Dh
PK2 = PK // 2
OW = 2 * Dh
LW = NL * P

NC, NS = 2, 16
NW = NC * NS
CHUNK = 128
NCK = N // CHUNK

CSB = 512
ATT_R = 2048
SUBBLK = {256: 256, 64: 128, 16: 128}


def _class_matrix(n_buckets: int) -> np.ndarray:
    bv = [(r * 9973) % n_buckets for r in range(P)]
    distinct = sorted(set(bv))
    cls = {v: i for i, v in enumerate(distinct)}
    m = np.zeros((P, P), np.float32)
    for r in range(P):
        m[r, cls[bv[r]]] = 1.0
    return m


QKV_RB = 512


def _qkv_body(x_ref, wq_ref, wk_ref, wv_ref, bqkv_ref, pk_ref, q_ref):
    xb = x_ref[0]
    ts = []
    for w_ref, bi in ((wq_ref, 0), (wk_ref, 1), (wv_ref, 2)):
        ts.append(jnp.dot(xb, w_ref[...],
                          preferred_element_type=jnp.float32)
                  + bqkv_ref[bi:bi + 1, :])
    tq, tk, tv = ts
    zpad = jnp.zeros((QKV_RB, Dh), jnp.float32)
    for h in range(H):
        sl = slice(h * Dh, (h + 1) * Dh)
        pk_ref[0, h] = jnp.concatenate(
            [tq[:, sl], tk[:, sl], tv[:, sl], zpad], axis=1)
        q_ref[0, h] = tq[:, sl]


def _qkv(x3, wqT, wkT, wvT, bqkv):
    grid = (B, N // QKV_RB)
    wspec = pl.BlockSpec((E, E), lambda b, i: (0, 0))
    xs = pl.BlockSpec((1, QKV_RB, E), lambda b, i: (b, i, 0))
    return pl.pallas_call(
        _qkv_body,
        grid=grid,
        in_specs=[xs, wspec, wspec, wspec,
                  pl.BlockSpec((NL, E), lambda b, i: (0, 0))],
        out_specs=[
            pl.BlockSpec((1, H, QKV_RB, PK), lambda b, i: (b, 0, i, 0)),
            pl.BlockSpec((1, H, QKV_RB, Dh), lambda b, i: (b, 0, i, 0)),
        ],
        out_shape=[
            jax.ShapeDtypeStruct((B, H, N, PK), jnp.float32),
            jax.ShapeDtypeStruct((B, H, N, Dh), jnp.float32),
        ],
    )(x3, wqT, wkT, wvT, bqkv)


def _rank_body(q_ref, probes_ref, m3_ref, tril_ref, su3f_ref, su3b_ref,
               ranks_ref, ohc_ref, run_ref):
    b = pl.program_id(0)
    h = pl.program_id(1)
    q2 = q_ref[0, 0]
    probes3 = jnp.concatenate([probes_ref[lvl, 0] for lvl in range(NL)],
                              axis=1)
    proj = jnp.dot(q2, probes3,
                   preferred_element_type=jnp.float32)
    mxs = [jnp.max(proj[:, lvl * P:(lvl + 1) * P], axis=1, keepdims=True)
           for lvl in range(NL)]
    mx3 = jnp.concatenate(
        [jnp.broadcast_to(m, (N, P)) for m in mxs], axis=1)
    eq3 = (proj == mx3).astype(jnp.bfloat16)
    cb3 = jnp.dot(eq3, su3b_ref[...],
                  preferred_element_type=jnp.float32)
    oh3 = jnp.where(cb3 == 0.0, eq3, jnp.bfloat16(0.0))
    ohc_ref[...] = jnp.dot(oh3, m3_ref[...],
                           preferred_element_type=jnp.float32)

    def blk(i, carry):
        seg = ohc_ref[pl.ds(i * CSB, CSB), :]
        run = jnp.dot(tril_ref[...], seg.astype(jnp.bfloat16),
                      preferred_element_type=jnp.float32) + carry
        run_ref[pl.ds(i * CSB, CSB), :] = run
        return run[CSB - 1:CSB, :]

    totals = lax.fori_loop(0, N // CSB, blk,
                           jnp.zeros((1, LW), jnp.float32))
    offs = jnp.dot(totals, su3f_ref[...], precision="highest",
                   preferred_element_type=jnp.float32)
    vals = ohc_ref[...] * (run_ref[...] + (offs - 1.0))
    base = (b * H + h) * N
    rows = []
    for lvl in range(NL):
        rankf = jnp.sum(vals[:, lvl * P:(lvl + 1) * P], axis=1)
        rows.append((rankf.astype(jnp.int32) + base).reshape(1, N))
    ranks_ref[0, 0] = jnp.concatenate(rows, axis=0)


def _ranks(q4, probes, m3, tril, su3f, su3b):
    return pl.pallas_call(
        _rank_body,
        grid=(B, H),
        in_specs=[
            pl.BlockSpec((1, 1, N, Dh), lambda b, h: (b, h, 0, 0)),
            pl.BlockSpec((NL, 1, Dh, P), lambda b, h: (0, h, 0, 0)),
            pl.BlockSpec((LW, LW), lambda b, h: (0, 0)),
            pl.BlockSpec((CSB, CSB), lambda b, h: (0, 0)),
            pl.BlockSpec((LW, LW), lambda b, h: (0, 0)),
            pl.BlockSpec((LW, LW), lambda b, h: (0, 0)),
        ],
        out_specs=pl.BlockSpec((1, 1, NL, N), lambda b, h: (b, h, 0, 0)),
        out_shape=jax.ShapeDtypeStruct((B, H, NL, N), jnp.int32),
        scratch_shapes=[
            pltpu.VMEM((N, LW), jnp.float32),
            pltpu.VMEM((N, LW), jnp.float32),
        ],
    )(q4, probes, m3, tril, su3f, su3b)


def _sc_sort_lvl(lvl, ranks_flat, pk2):
    mesh = plsc.VectorSubcoreMesh(core_axis_name="c", subcore_axis_name="s")
    n_per = BH * NCK // NW

    @functools.partial(
        pl.kernel, mesh=mesh,
        out_type=jax.ShapeDtypeStruct((BH * N, PK), jnp.float32),
        scratch_types=[
            pltpu.VMEM((2, CHUNK), jnp.int32),
            pltpu.VMEM((CHUNK, PK), jnp.float32),
            pltpu.VMEM((CHUNK, PK), jnp.float32),
            pltpu.SemaphoreType.DMA,
            pltpu.SemaphoreType.DMA,
        ],
        name=f"sc_sort_l{lvl}",
    )
    def body(ranks_h, pk_h, dst, idx2, rows0, rows1, sem0, sem1):
        wid = lax.axis_index("s") * NC + lax.axis_index("c")
        rbufs = (rows0, rows1)
        sems = (sem0, sem1)

        def pair(p, _):
            for j in range(2):
                i = p * 2 + j
                t = wid * n_per + i
                bh = t // NCK
                ck = t - bh * NCK
                n0 = ck * CHUNK
                row0 = bh * N + n0
                roff = (bh * NL + lvl) * N + n0
                @pl.when(p > 0)
                def _(j=j):
                    pltpu.make_async_copy(
                        rbufs[j], dst.at[idx2.at[j]], sems[j]).wait()
                pltpu.sync_copy(ranks_h.at[pl.ds(roff, CHUNK)],
                                idx2.at[j])
                pltpu.sync_copy(pk_h.at[pl.ds(row0, CHUNK)], rbufs[j])
                pltpu.async_copy(rbufs[j], dst.at[idx2.at[j]], sems[j])
            return 0

        lax.fori_loop(0, n_per // 2, pair, 0)
        pltpu.make_async_copy(rows0, dst.at[idx2.at[0]], sem0).wait()
        pltpu.make_async_copy(rows1, dst.at[idx2.at[1]], sem1).wait()

    return body(ranks_flat, pk2)


def _att_body(cs, r, sb, m_ref, halo_ref, e_ref,
              band_ref, c0_ref, o_ref):
    g = pl.program_id(1)
    nb = r // sb
    kb = sb + cs
    pkb = m_ref[0].astype(jnp.bfloat16)
    hb = halo_ref[0].astype(jnp.bfloat16)
    def sel(src_b, c):
        return jnp.dot(src_b, e_ref[c],
                       preferred_element_type=jnp.float32
                       ).astype(jnp.bfloat16)
    qb, kc, vc = sel(pkb, 0), sel(pkb, 1), sel(pkb, 2)
    kh, vh = sel(hb, 1), sel(hb, 2)
    kwin = jnp.concatenate([kh, kc], axis=0)
    vwin = jnp.concatenate([vh, vc], axis=0)
    q3 = qb.reshape(nb, sb, Dh)
    k3 = jnp.concatenate(
        [kwin[s * sb:s * sb + kb].reshape(1, kb, Dh) for s in range(nb)],
        axis=0)
    v3 = jnp.concatenate(
        [vwin[s * sb:s * sb + kb].reshape(1, kb, Dh) for s in range(nb)],
        axis=0)
    scores = lax.dot_general(
        q3, k3, (((2,), (2,)), ((0,), (0,))),
        preferred_element_type=jnp.float32)
    first = (lax.broadcasted_iota(jnp.int32, (nb, 1, 1), 0) == 0) & (g == 0)
    scores = scores + band_ref[...] + jnp.where(
        first, 1.0, 0.0) * c0_ref[...]
    m = jnp.max(scores, axis=2, keepdims=True)
    e = jnp.exp(scores - m)
    s = jnp.sum(e, axis=2, keepdims=True)
    attn = (e / s).astype(jnp.bfloat16)
    out = lax.dot_general(attn, v3, (((2,), (1,)), ((0,), (0,))),
                          preferred_element_type=jnp.float32)
    o_ref[0] = jnp.concatenate(
        [out.reshape(r, Dh), jnp.zeros((r, OW - Dh), jnp.float32)],
        axis=1)


def _band_masks(cs, sb):
    kb = sb + cs
    ci = np.arange(sb)[:, None] // cs
    kc = np.arange(kb)[None, :] // cs - 1
    valid = (kc == ci) | (kc == ci - 1)
    band = np.where(valid, 0.0, -1e9).astype(np.float32)[None]
    c0 = np.where((kc == -1) & valid, -1e9, 0.0).astype(np.float32)[None]
    return band, c0


def _esel():
    e = np.zeros((3, PK, Dh), np.float32)
    for c in range(3):
        for i in range(Dh):
            e[c, c * Dh + i, i] = 1.0
    e[0] *= 1.0 / math.sqrt(Dh)
    return jnp.asarray(e).astype(jnp.bfloat16)


def _attention(cs, sorted3):
    r, sb = ATT_R, SUBBLK[cs]
    nb, kb = r // sb, sb + cs
    band, c0 = _band_masks(cs, sb)
    grid = (BH, N // r)
    gg = r // cs
    mspec = pl.BlockSpec((1, r, PK), lambda bh, g: (bh, g, 0))
    halo = pl.BlockSpec((1, cs, PK),
                        lambda bh, g: (bh, jnp.maximum(g * gg - 1, 0), 0))
    out = pl.pallas_call(
        functools.partial(_att_body, cs, r, sb),
        grid=grid,
        in_specs=[mspec, halo,
                  pl.BlockSpec((3, PK, Dh), lambda bh, g: (0, 0, 0)),
                  pl.BlockSpec((1, sb, kb), lambda bh, g: (0, 0, 0)),
                  pl.BlockSpec((1, sb, kb), lambda bh, g: (0, 0, 0))],
        out_specs=pl.BlockSpec((1, r, OW), lambda bh, g: (bh, g, 0)),
        out_shape=jax.ShapeDtypeStruct((BH, N, OW), jnp.float32),
    )(sorted3, sorted3, _esel(), jnp.asarray(band), jnp.asarray(c0))
    return out.reshape(BH * N, OW)


def _sc_unsort_lvl(lvl, ranks_flat, o_l):
    mesh = plsc.VectorSubcoreMesh(core_axis_name="c", subcore_axis_name="s")
    n_per = BH * NCK // NW

    @functools.partial(
        pl.kernel, mesh=mesh,
        out_type=jax.ShapeDtypeStruct((BH * N, OW), jnp.float32),
        scratch_types=[
            pltpu.VMEM((2, CHUNK), jnp.int32),
            pltpu.VMEM((CHUNK, OW), jnp.float32),
            pltpu.VMEM((CHUNK, OW), jnp.float32),
            pltpu.SemaphoreType.DMA,
        ],
        name=f"sc_unsort_l{lvl}",
    )
    def body(ranks_h, o_h, u_h, idx2, rows0, rows1, sem):
        wid = lax.axis_index("s") * NC + lax.axis_index("c")
        rbufs = (rows0, rows1)

        def pair(p, _):
            for j in range(2):
                i = p * 2 + j
                t = wid * n_per + i
                bh = t // NCK
                ck = t - bh * NCK
                n0 = ck * CHUNK
                roff = (bh * NL + lvl) * N + n0
                pltpu.sync_copy(ranks_h.at[pl.ds(roff, CHUNK)],
                                idx2.at[j])
                pltpu.async_copy(o_h.at[idx2.at[j]], rbufs[j], sem).wait()
                pltpu.sync_copy(rbufs[j],
                                u_h.at[pl.ds(bh * N + n0, CHUNK)])
            return 0

        lax.fori_loop(0, n_per // 2, pair, 0)

    return body(ranks_flat, o_l)


def _out_body(u0_ref, u1_ref, u2_ref, woT_ref, bo_ref, o_ref):
    parts = []
    for h in range(H):
        s = (u0_ref[0, h, :, :Dh] + u1_ref[0, h, :, :Dh]
             + u2_ref[0, h, :, :Dh])
        parts.append(s)
    m2 = jnp.concatenate(parts, axis=1)
    o_ref[0] = (
        jnp.dot(m2, woT_ref[...],
                preferred_element_type=jnp.float32) + bo_ref[...])


def _oproj(u0, u1, u2, woT3, bo):
    rb = 512
    uspec = pl.BlockSpec((1, H, rb, OW), lambda b, i: (b, 0, i, 0))
    out = pl.pallas_call(
        _out_body,
        grid=(B, N // rb),
        in_specs=[uspec, uspec, uspec,
                  pl.BlockSpec((E, E), lambda b, i: (0, 0)),
                  pl.BlockSpec((1, E), lambda b, i: (0, 0))],
        out_specs=pl.BlockSpec((1, rb, E), lambda b, i: (b, i, 0)),
        out_shape=jax.ShapeDtypeStruct((B, N, E), jnp.float32),
    )(u0.reshape(B, H, N, OW), u1.reshape(B, H, N, OW),
      u2.reshape(B, H, N, OW), woT3, bo.reshape(1, E))
    return out


def kernel(x, Wq, bq, Wk, bk, Wv, bv, Wo, bo, probes):
    bqkv = jnp.stack([bq, bk, bv], axis=0)
    pk4, q4 = _qkv(x, Wq.T, Wk.T, Wv.T, bqkv)

    m3np = np.zeros((LW, LW), np.float32)
    su3np = np.zeros((LW, LW), np.float32)
    for lvl, cs in enumerate(BUCKET_SIZES):
        s = slice(lvl * P, (lvl + 1) * P)
        m3np[s, s] = _class_matrix(max(1, N // cs))
        su3np[s, s] = np.triu(np.ones((P, P), np.float32), 1)
    m3 = jnp.asarray(m3np).astype(jnp.bfloat16)
    su3f = jnp.asarray(su3np)
    su3b = jnp.asarray(su3np).astype(jnp.bfloat16)
    tril = jnp.asarray(np.tril(np.ones((CSB, CSB), np.float32))
                       ).astype(jnp.bfloat16)
    ranks = _ranks(q4, probes, m3, tril, su3f, su3b)
    ranks_flat = ranks.reshape(-1)

    pk2 = pk4.reshape(BH * N, PK)
    sorted_lvls = [_sc_sort_lvl(lvl, ranks_flat, pk2) for lvl in range(NL)]
    outs = [_attention(cs, sorted_lvls[lvl].reshape(BH, N, PK))
            for lvl, cs in enumerate(BUCKET_SIZES)]
    us = [_sc_unsort_lvl(lvl, ranks_flat, outs[lvl]) for lvl in range(NL)]
    return _oproj(us[0], us[1], us[2], Wo.T * (1.0 / NL), bo)


# speedup vs baseline: 1.8696x; 1.8696x over previous
import functools
import math

import numpy as np
import jax
import jax.numpy as jnp
from jax import lax
from jax.experimental import pallas as pl
from jax.experimental.pallas import tpu as pltpu
from jax.experimental.pallas import tpu_sc as plsc

B, N, E = 2, 8192, 768
H = 12
Dh = E // H
P = 64
BUCKET_SIZES = (256, 64, 16)
NL = len(BUCKET_SIZES)
BH = B * H
PK = 4 * Dh
PK2 = PK // 2
OW = 2 * Dh
LW = NL * P

NC, NS = 2, 16
NW = NC * NS
CHUNK = 128
NCK = N // CHUNK

CSB = 512
ATT_R = 4096
SUBBLK = {256: 256, 64: 128, 16: 128}


def _class_matrix(n_buckets: int) -> np.ndarray:
    bv = [(r * 9973) % n_buckets for r in range(P)]
    distinct = sorted(set(bv))
    cls = {v: i for i, v in enumerate(distinct)}
    m = np.zeros((P, P), np.float32)
    for r in range(P):
        m[r, cls[bv[r]]] = 1.0
    return m


QKV_RB = 512


def _qkv_body(x_ref, wq_ref, wk_ref, wv_ref, bqkv_ref, pk_ref, q_ref):
    xb = x_ref[0]
    ts = []
    for w_ref, bi in ((wq_ref, 0), (wk_ref, 1), (wv_ref, 2)):
        ts.append(jnp.dot(xb, w_ref[...],
                          preferred_element_type=jnp.float32)
                  + bqkv_ref[bi:bi + 1, :])
    tq, tk, tv = ts
    zpad = jnp.zeros((QKV_RB, Dh), jnp.float32)
    for h in range(H):
        sl = slice(h * Dh, (h + 1) * Dh)
        pk_ref[0, h] = jnp.concatenate(
            [tq[:, sl], tk[:, sl], tv[:, sl], zpad], axis=1)
        q_ref[0, h] = tq[:, sl]


def _qkv(x3, wqT, wkT, wvT, bqkv):
    grid = (B, N // QKV_RB)
    wspec = pl.BlockSpec((E, E), lambda b, i: (0, 0))
    xs = pl.BlockSpec((1, QKV_RB, E), lambda b, i: (b, i, 0))
    return pl.pallas_call(
        _qkv_body,
        grid=grid,
        in_specs=[xs, wspec, wspec, wspec,
                  pl.BlockSpec((NL, E), lambda b, i: (0, 0))],
        out_specs=[
            pl.BlockSpec((1, H, QKV_RB, PK), lambda b, i: (b, 0, i, 0)),
            pl.BlockSpec((1, H, QKV_RB, Dh), lambda b, i: (b, 0, i, 0)),
        ],
        out_shape=[
            jax.ShapeDtypeStruct((B, H, N, PK), jnp.float32),
            jax.ShapeDtypeStruct((B, H, N, Dh), jnp.float32),
        ],
    )(x3, wqT, wkT, wvT, bqkv)


def _rank_body(q_ref, probes_ref, m3_ref, tril_ref, su3f_ref, su3b_ref,
               ranks_ref, ohc_ref, run_ref):
    b = pl.program_id(0)
    h = pl.program_id(1)
    q2 = q_ref[0, 0]
    probes3 = jnp.concatenate([probes_ref[lvl, 0] for lvl in range(NL)],
                              axis=1)
    proj = jnp.dot(q2, probes3,
                   preferred_element_type=jnp.float32)
    mxs = [jnp.max(proj[:, lvl * P:(lvl + 1) * P], axis=1, keepdims=True)
           for lvl in range(NL)]
    mx3 = jnp.concatenate(
        [jnp.broadcast_to(m, (N, P)) for m in mxs], axis=1)
    eq3 = (proj == mx3).astype(jnp.bfloat16)
    cb3 = jnp.dot(eq3, su3b_ref[...],
                  preferred_element_type=jnp.float32)
    oh3 = jnp.where(cb3 == 0.0, eq3, jnp.bfloat16(0.0))
    ohc_ref[...] = jnp.dot(oh3, m3_ref[...],
                           preferred_element_type=jnp.float32)

    def blk(i, carry):
        seg = ohc_ref[pl.ds(i * CSB, CSB), :]
        run = jnp.dot(tril_ref[...], seg.astype(jnp.bfloat16),
                      preferred_element_type=jnp.float32) + carry
        run_ref[pl.ds(i * CSB, CSB), :] = run
        return run[CSB - 1:CSB, :]

    totals = lax.fori_loop(0, N // CSB, blk,
                           jnp.zeros((1, LW), jnp.float32))
    offs = jnp.dot(totals, su3f_ref[...], precision="highest",
                   preferred_element_type=jnp.float32)
    vals = ohc_ref[...] * (run_ref[...] + (offs - 1.0))
    base = (b * H + h) * N
    rows = []
    for lvl in range(NL):
        rankf = jnp.sum(vals[:, lvl * P:(lvl + 1) * P], axis=1)
        rows.append((rankf.astype(jnp.int32) + base).reshape(1, N))
    ranks_ref[0, 0] = jnp.concatenate(rows, axis=0)


def _ranks(q4, probes, m3, tril, su3f, su3b):
    return pl.pallas_call(
        _rank_body,
        grid=(B, H),
        in_specs=[
            pl.BlockSpec((1, 1, N, Dh), lambda b, h: (b, h, 0, 0)),
            pl.BlockSpec((NL, 1, Dh, P), lambda b, h: (0, h, 0, 0)),
            pl.BlockSpec((LW, LW), lambda b, h: (0, 0)),
            pl.BlockSpec((CSB, CSB), lambda b, h: (0, 0)),
            pl.BlockSpec((LW, LW), lambda b, h: (0, 0)),
            pl.BlockSpec((LW, LW), lambda b, h: (0, 0)),
        ],
        out_specs=pl.BlockSpec((1, 1, NL, N), lambda b, h: (b, h, 0, 0)),
        out_shape=jax.ShapeDtypeStruct((B, H, NL, N), jnp.int32),
        scratch_shapes=[
            pltpu.VMEM((N, LW), jnp.float32),
            pltpu.VMEM((N, LW), jnp.float32),
        ],
    )(q4, probes, m3, tril, su3f, su3b)


def _sc_sort_lvl(lvl, ranks_flat, pk2):
    mesh = plsc.VectorSubcoreMesh(core_axis_name="c", subcore_axis_name="s")
    n_per = BH * NCK // NW

    @functools.partial(
        pl.kernel, mesh=mesh,
        out_type=jax.ShapeDtypeStruct((BH * N, PK), jnp.float32),
        scratch_types=[
            pltpu.VMEM((2, CHUNK), jnp.int32),
            pltpu.VMEM((CHUNK, PK), jnp.float32),
            pltpu.VMEM((CHUNK, PK), jnp.float32),
            pltpu.SemaphoreType.DMA,
            pltpu.SemaphoreType.DMA,
        ],
        name=f"sc_sort_l{lvl}",
    )
    def body(ranks_h, pk_h, dst, idx2, rows0, rows1, sem0, sem1):
        wid = lax.axis_index("s") * NC + lax.axis_index("c")
        rbufs = (rows0, rows1)
        sems = (sem0, sem1)

        def pair(p, _):
            for j in range(2):
                i = p * 2 + j
                t = wid * n_per + i
                bh = t // NCK
                ck = t - bh * NCK
                n0 = ck * CHUNK
                row0 = bh * N + n0
                roff = (bh * NL + lvl) * N + n0
                @pl.when(p > 0)
                def _(j=j):
                    pltpu.make_async_copy(
                        rbufs[j], dst.at[idx2.at[j]], sems[j]).wait()
                pltpu.sync_copy(ranks_h.at[pl.ds(roff, CHUNK)],
                                idx2.at[j])
                pltpu.sync_copy(pk_h.at[pl.ds(row0, CHUNK)], rbufs[j])
                pltpu.async_copy(rbufs[j], dst.at[idx2.at[j]], sems[j])
            return 0

        lax.fori_loop(0, n_per // 2, pair, 0)
        pltpu.make_async_copy(rows0, dst.at[idx2.at[0]], sem0).wait()
        pltpu.make_async_copy(rows1, dst.at[idx2.at[1]], sem1).wait()

    return body(ranks_flat, pk2)


def _att_body(cs, r, sb, m_ref, halo_ref, e_ref,
              band_ref, c0_ref, o_ref):
    g = pl.program_id(1)
    nb = r // sb
    kb = sb + cs
    pkb = m_ref[0].astype(jnp.bfloat16)
    hb = halo_ref[0].astype(jnp.bfloat16)
    def sel(src_b, c):
        return jnp.dot(src_b, e_ref[c],
                       preferred_element_type=jnp.float32
                       ).astype(jnp.bfloat16)
    qb, kc, vc = sel(pkb, 0), sel(pkb, 1), sel(pkb, 2)
    kh, vh = sel(hb, 1), sel(hb, 2)
    kwin = jnp.concatenate([kh, kc], axis=0)
    vwin = jnp.concatenate([vh, vc], axis=0)
    q3 = qb.reshape(nb, sb, Dh)
    k3 = jnp.concatenate(
        [kwin[s * sb:s * sb + kb].reshape(1, kb, Dh) for s in range(nb)],
        axis=0)
    v3 = jnp.concatenate(
        [vwin[s * sb:s * sb + kb].reshape(1, kb, Dh) for s in range(nb)],
        axis=0)
    scores = lax.dot_general(
        q3, k3, (((2,), (2,)), ((0,), (0,))),
        preferred_element_type=jnp.float32)
    first = (lax.broadcasted_iota(jnp.int32, (nb, 1, 1), 0) == 0) & (g == 0)
    scores = scores + band_ref[...] + jnp.where(
        first, 1.0, 0.0) * c0_ref[...]
    e = jnp.exp(scores)
    s = jnp.sum(e, axis=2, keepdims=True)
    attn = (e / s).astype(jnp.bfloat16)
    out = lax.dot_general(attn, v3, (((2,), (1,)), ((0,), (0,))),
                          preferred_element_type=jnp.float32)
    o_ref[0] = jnp.concatenate(
        [out.reshape(r, Dh), jnp.zeros((r, OW - Dh), jnp.float32)],
        axis=1)


def _band_masks(cs, sb):
    kb = sb + cs
    ci = np.arange(sb)[:, None] // cs
    kc = np.arange(kb)[None, :] // cs - 1
    valid = (kc == ci) | (kc == ci - 1)
    band = np.where(valid, 0.0, -1e9).astype(np.float32)[None]
    c0 = np.where((kc == -1) & valid, -1e9, 0.0).astype(np.float32)[None]
    return band, c0


def _esel():
    e = np.zeros((3, PK, Dh), np.float32)
    for c in range(3):
        for i in range(Dh):
            e[c, c * Dh + i, i] = 1.0
    e[0] *= 1.0 / math.sqrt(Dh)
    return jnp.asarray(e).astype(jnp.bfloat16)


def _attention(cs, sorted3):
    r, sb = ATT_R, SUBBLK[cs]
    nb, kb = r // sb, sb + cs
    band, c0 = _band_masks(cs, sb)
    grid = (BH, N // r)
    gg = r // cs
    mspec = pl.BlockSpec((1, r, PK), lambda bh, g: (bh, g, 0))
    halo = pl.BlockSpec((1, cs, PK),
                        lambda bh, g: (bh, jnp.maximum(g * gg - 1, 0), 0))
    out = pl.pallas_call(
        functools.partial(_att_body, cs, r, sb),
        grid=grid,
        in_specs=[mspec, halo,
                  pl.BlockSpec((3, PK, Dh), lambda bh, g: (0, 0, 0)),
                  pl.BlockSpec((1, sb, kb), lambda bh, g: (0, 0, 0)),
                  pl.BlockSpec((1, sb, kb), lambda bh, g: (0, 0, 0))],
        out_specs=pl.BlockSpec((1, r, OW), lambda bh, g: (bh, g, 0)),
        out_shape=jax.ShapeDtypeStruct((BH, N, OW), jnp.float32),
    )(sorted3, sorted3, _esel(), jnp.asarray(band), jnp.asarray(c0))
    return out.reshape(BH * N, OW)


def _sc_unsort_lvl(lvl, ranks_flat, o_l):
    mesh = plsc.VectorSubcoreMesh(core_axis_name="c", subcore_axis_name="s")
    n_per = BH * NCK // NW

    @functools.partial(
        pl.kernel, mesh=mesh,
        out_type=jax.ShapeDtypeStruct((BH * N, OW), jnp.float32),
        scratch_types=[
            pltpu.VMEM((2, CHUNK), jnp.int32),
            pltpu.VMEM((CHUNK, OW), jnp.float32),
            pltpu.VMEM((CHUNK, OW), jnp.float32),
            pltpu.SemaphoreType.DMA,
        ],
        name=f"sc_unsort_l{lvl}",
    )
    def body(ranks_h, o_h, u_h, idx2, rows0, rows1, sem):
        wid = lax.axis_index("s") * NC + lax.axis_index("c")
        rbufs = (rows0, rows1)

        def pair(p, _):
            for j in range(2):
                i = p * 2 + j
                t = wid * n_per + i
                bh = t // NCK
                ck = t - bh * NCK
                n0 = ck * CHUNK
                roff = (bh * NL + lvl) * N + n0
                pltpu.sync_copy(ranks_h.at[pl.ds(roff, CHUNK)],
                                idx2.at[j])
                pltpu.async_copy(o_h.at[idx2.at[j]], rbufs[j], sem).wait()
                pltpu.sync_copy(rbufs[j],
                                u_h.at[pl.ds(bh * N + n0, CHUNK)])
            return 0

        lax.fori_loop(0, n_per // 2, pair, 0)

    return body(ranks_flat, o_l)


def _out_body(u0_ref, u1_ref, u2_ref, woT_ref, bo_ref, o_ref):
    parts = []
    for h in range(H):
        s = (u0_ref[0, h, :, :Dh] + u1_ref[0, h, :, :Dh]
             + u2_ref[0, h, :, :Dh])
        parts.append(s)
    m2 = jnp.concatenate(parts, axis=1).astype(jnp.bfloat16)
    o_ref[0] = (
        jnp.dot(m2, woT_ref[...].astype(jnp.bfloat16),
                preferred_element_type=jnp.float32) + bo_ref[...])


def _oproj(u0, u1, u2, woT3, bo):
    rb = 512
    uspec = pl.BlockSpec((1, H, rb, OW), lambda b, i: (b, 0, i, 0))
    out = pl.pallas_call(
        _out_body,
        grid=(B, N // rb),
        in_specs=[uspec, uspec, uspec,
                  pl.BlockSpec((E, E), lambda b, i: (0, 0)),
                  pl.BlockSpec((1, E), lambda b, i: (0, 0))],
        out_specs=pl.BlockSpec((1, rb, E), lambda b, i: (b, i, 0)),
        out_shape=jax.ShapeDtypeStruct((B, N, E), jnp.float32),
    )(u0.reshape(B, H, N, OW), u1.reshape(B, H, N, OW),
      u2.reshape(B, H, N, OW), woT3, bo.reshape(1, E))
    return out


def kernel(x, Wq, bq, Wk, bk, Wv, bv, Wo, bo, probes):
    bqkv = jnp.stack([bq, bk, bv], axis=0)
    pk4, q4 = _qkv(x, Wq.T, Wk.T, Wv.T, bqkv)

    m3np = np.zeros((LW, LW), np.float32)
    su3np = np.zeros((LW, LW), np.float32)
    for lvl, cs in enumerate(BUCKET_SIZES):
        s = slice(lvl * P, (lvl + 1) * P)
        m3np[s, s] = _class_matrix(max(1, N // cs))
        su3np[s, s] = np.triu(np.ones((P, P), np.float32), 1)
    m3 = jnp.asarray(m3np).astype(jnp.bfloat16)
    su3f = jnp.asarray(su3np)
    su3b = jnp.asarray(su3np).astype(jnp.bfloat16)
    tril = jnp.asarray(np.tril(np.ones((CSB, CSB), np.float32))
                       ).astype(jnp.bfloat16)
    ranks = _ranks(q4, probes, m3, tril, su3f, su3b)
    ranks_flat = ranks.reshape(-1)

    pk2 = pk4.reshape(BH * N, PK)
    sorted_lvls = [_sc_sort_lvl(lvl, ranks_flat, pk2) for lvl in range(NL)]
    outs = [_attention(cs, sorted_lvls[lvl].reshape(BH, N, PK))
            for lvl, cs in enumerate(BUCKET_SIZES)]
    us = [_sc_unsort_lvl(lvl, ranks_flat, outs[lvl]) for lvl in range(NL)]
    return _oproj(us[0], us[1], us[2], Wo.T * (1.0 / NL), bo)


# speedup vs baseline: 1.9211x; 1.0275x over previous
import functools
import math

import numpy as np
import jax
import jax.numpy as jnp
from jax import lax
from jax.experimental import pallas as pl
from jax.experimental.pallas import tpu as pltpu
from jax.experimental.pallas import tpu_sc as plsc

B, N, E = 2, 8192, 768
H = 12
Dh = E // H
P = 64
BUCKET_SIZES = (256, 64, 16)
NL = len(BUCKET_SIZES)
BH = B * H
PK = 4 * Dh
PK2 = PK // 2
OW = 2 * Dh
LW = NL * P

NC, NS = 2, 16
NW = NC * NS
CHUNK = 128
NCK = N // CHUNK

CSB = 512
ATT_R = 4096
SUBBLK = {256: 256, 64: 128, 16: 128}


def _class_matrix(n_buckets: int) -> np.ndarray:
    bv = [(r * 9973) % n_buckets for r in range(P)]
    distinct = sorted(set(bv))
    cls = {v: i for i, v in enumerate(distinct)}
    m = np.zeros((P, P), np.float32)
    for r in range(P):
        m[r, cls[bv[r]]] = 1.0
    return m


QKV_RB = 512


def _qkv_body(x_ref, wq_ref, wk_ref, wv_ref, bqkv_ref, pk_ref, q_ref):
    xb = x_ref[0]
    ts = []
    for w_ref, bi in ((wq_ref, 0), (wk_ref, 1), (wv_ref, 2)):
        ts.append(jnp.dot(xb, w_ref[...],
                          preferred_element_type=jnp.float32)
                  + bqkv_ref[bi:bi + 1, :])
    tq, tk, tv = ts
    zpad = jnp.zeros((QKV_RB, Dh), jnp.float32)
    for h in range(H):
        sl = slice(h * Dh, (h + 1) * Dh)
        pk_ref[0, h] = jnp.concatenate(
            [tq[:, sl], tk[:, sl], tv[:, sl], zpad], axis=1)
        q_ref[0, h] = tq[:, sl]


def _qkv(x3, wqT, wkT, wvT, bqkv):
    grid = (B, N // QKV_RB)
    wspec = pl.BlockSpec((E, E), lambda b, i: (0, 0))
    xs = pl.BlockSpec((1, QKV_RB, E), lambda b, i: (b, i, 0))
    return pl.pallas_call(
        _qkv_body,
        grid=grid,
        in_specs=[xs, wspec, wspec, wspec,
                  pl.BlockSpec((NL, E), lambda b, i: (0, 0))],
        out_specs=[
            pl.BlockSpec((1, H, QKV_RB, PK), lambda b, i: (b, 0, i, 0)),
            pl.BlockSpec((1, H, QKV_RB, Dh), lambda b, i: (b, 0, i, 0)),
        ],
        out_shape=[
            jax.ShapeDtypeStruct((B, H, N, PK), jnp.float32),
            jax.ShapeDtypeStruct((B, H, N, Dh), jnp.float32),
        ],
    )(x3, wqT, wkT, wvT, bqkv)


def _rank_body(q_ref, probes_ref, m3_ref, tril_ref, su3f_ref, su3b_ref,
               sel_ref, ranks_ref, ohc_ref, run_ref):
    b = pl.program_id(0)
    h = pl.program_id(1)
    q2 = q_ref[0, 0]
    probes3 = jnp.concatenate([probes_ref[lvl, 0] for lvl in range(NL)],
                              axis=1)
    proj = jnp.dot(q2, probes3,
                   preferred_element_type=jnp.float32)
    mxs = [jnp.max(proj[:, lvl * P:(lvl + 1) * P], axis=1, keepdims=True)
           for lvl in range(NL)]
    mx3 = jnp.concatenate(
        [jnp.broadcast_to(m, (N, P)) for m in mxs], axis=1)
    eq3 = (proj == mx3).astype(jnp.bfloat16)
    cb3 = jnp.dot(eq3, su3b_ref[...],
                  preferred_element_type=jnp.float32)
    oh3 = jnp.where(cb3 == 0.0, eq3, jnp.bfloat16(0.0))
    ohc_ref[...] = jnp.dot(oh3, m3_ref[...],
                           preferred_element_type=jnp.float32)

    def blk(i, carry):
        seg = ohc_ref[pl.ds(i * CSB, CSB), :]
        run = jnp.dot(tril_ref[...], seg.astype(jnp.bfloat16),
                      preferred_element_type=jnp.float32) + carry
        run_ref[pl.ds(i * CSB, CSB), :] = run
        return run[CSB - 1:CSB, :]

    totals = lax.fori_loop(0, N // CSB, blk,
                           jnp.zeros((1, LW), jnp.float32))
    offs = jnp.dot(totals, su3f_ref[...], precision="highest",
                   preferred_element_type=jnp.float32)
    vals = ohc_ref[...] * (run_ref[...] + (offs - 1.0))
    base = (b * H + h) * N
    hi = jnp.floor(vals * (1.0 / 256.0))
    lo = vals - hi * 256.0
    dims = (((1,), (1,)), ((), ()))
    sel_b = sel_ref[...].astype(jnp.bfloat16)
    r_hi = lax.dot_general(sel_b, hi.astype(jnp.bfloat16), dims,
                           preferred_element_type=jnp.float32)
    r_lo = lax.dot_general(sel_b, lo.astype(jnp.bfloat16), dims,
                           preferred_element_type=jnp.float32)
    ranks_ref[0, 0] = (r_hi * 256.0 + r_lo).astype(jnp.int32) + base


def _ranks(q4, probes, m3, tril, su3f, su3b, sel3):
    return pl.pallas_call(
        _rank_body,
        grid=(B, H),
        in_specs=[
            pl.BlockSpec((1, 1, N, Dh), lambda b, h: (b, h, 0, 0)),
            pl.BlockSpec((NL, 1, Dh, P), lambda b, h: (0, h, 0, 0)),
            pl.BlockSpec((LW, LW), lambda b, h: (0, 0)),
            pl.BlockSpec((CSB, CSB), lambda b, h: (0, 0)),
            pl.BlockSpec((LW, LW), lambda b, h: (0, 0)),
            pl.BlockSpec((LW, LW), lambda b, h: (0, 0)),
            pl.BlockSpec((NL, LW), lambda b, h: (0, 0)),
        ],
        out_specs=pl.BlockSpec((1, 1, NL, N), lambda b, h: (b, h, 0, 0)),
        out_shape=jax.ShapeDtypeStruct((B, H, NL, N), jnp.int32),
        scratch_shapes=[
            pltpu.VMEM((N, LW), jnp.float32),
            pltpu.VMEM((N, LW), jnp.float32),
        ],
    )(q4, probes, m3, tril, su3f, su3b, sel3)


def _sc_sort_lvl(lvl, ranks_flat, pk2):
    mesh = plsc.VectorSubcoreMesh(core_axis_name="c", subcore_axis_name="s")
    n_per = BH * NCK // NW

    @functools.partial(
        pl.kernel, mesh=mesh,
        out_type=jax.ShapeDtypeStruct((BH * N, PK), jnp.float32),
        scratch_types=[
            pltpu.VMEM((2, CHUNK), jnp.int32),
            pltpu.VMEM((CHUNK, PK), jnp.float32),
            pltpu.VMEM((CHUNK, PK), jnp.float32),
            pltpu.SemaphoreType.DMA,
            pltpu.SemaphoreType.DMA,
        ],
        name=f"sc_sort_l{lvl}",
    )
    def body(ranks_h, pk_h, dst, idx2, rows0, rows1, sem0, sem1):
        wid = lax.axis_index("s") * NC + lax.axis_index("c")
        rbufs = (rows0, rows1)
        sems = (sem0, sem1)

        def pair(p, _):
            for j in range(2):
                i = p * 2 + j
                t = wid * n_per + i
                bh = t // NCK
                ck = t - bh * NCK
                n0 = ck * CHUNK
                row0 = bh * N + n0
                roff = (bh * NL + lvl) * N + n0
                @pl.when(p > 0)
                def _(j=j):
                    pltpu.make_async_copy(
                        rbufs[j], dst.at[idx2.at[j]], sems[j]).wait()
                pltpu.sync_copy(ranks_h.at[pl.ds(roff, CHUNK)],
                                idx2.at[j])
                pltpu.sync_copy(pk_h.at[pl.ds(row0, CHUNK)], rbufs[j])
                pltpu.async_copy(rbufs[j], dst.at[idx2.at[j]], sems[j])
            return 0

        lax.fori_loop(0, n_per // 2, pair, 0)
        pltpu.make_async_copy(rows0, dst.at[idx2.at[0]], sem0).wait()
        pltpu.make_async_copy(rows1, dst.at[idx2.at[1]], sem1).wait()

    return body(ranks_flat, pk2)


def _att_body(cs, r, sb, m_ref, halo_ref, e_ref,
              band_ref, c0_ref, o_ref):
    g = pl.program_id(1)
    nb = r // sb
    kb = sb + cs
    pkb = m_ref[0].astype(jnp.bfloat16)
    hb = halo_ref[0].astype(jnp.bfloat16)
    def sel(src_b, c):
        return jnp.dot(src_b, e_ref[c],
                       preferred_element_type=jnp.float32
                       ).astype(jnp.bfloat16)
    qb, kc, vc = sel(pkb, 0), sel(pkb, 1), sel(pkb, 2)
    kh, vh = sel(hb, 1), sel(hb, 2)
    kwin = jnp.concatenate([kh, kc], axis=0)
    vwin = jnp.concatenate([vh, vc], axis=0)
    q3 = qb.reshape(nb, sb, Dh)
    k3 = jnp.concatenate(
        [kwin[s * sb:s * sb + kb].reshape(1, kb, Dh) for s in range(nb)],
        axis=0)
    v3 = jnp.concatenate(
        [vwin[s * sb:s * sb + kb].reshape(1, kb, Dh) for s in range(nb)],
        axis=0)
    scores = lax.dot_general(
        q3, k3, (((2,), (2,)), ((0,), (0,))),
        preferred_element_type=jnp.float32)
    first = (lax.broadcasted_iota(jnp.int32, (nb, 1, 1), 0) == 0) & (g == 0)
    scores = scores + band_ref[...] + jnp.where(
        first, 1.0, 0.0) * c0_ref[...]
    e = jnp.exp(scores)
    s = jnp.sum(e, axis=2, keepdims=True)
    attn = (e / s).astype(jnp.bfloat16)
    out = lax.dot_general(attn, v3, (((2,), (1,)), ((0,), (0,))),
                          preferred_element_type=jnp.float32)
    o_ref[0] = jnp.concatenate(
        [out.reshape(r, Dh), jnp.zeros((r, OW - Dh), jnp.float32)],
        axis=1)


def _band_masks(cs, sb):
    kb = sb + cs
    ci = np.arange(sb)[:, None] // cs
    kc = np.arange(kb)[None, :] // cs - 1
    valid = (kc == ci) | (kc == ci - 1)
    band = np.where(valid, 0.0, -1e9).astype(np.float32)[None]
    c0 = np.where((kc == -1) & valid, -1e9, 0.0).astype(np.float32)[None]
    return band, c0


def _esel():
    e = np.zeros((3, PK, Dh), np.float32)
    for c in range(3):
        for i in range(Dh):
            e[c, c * Dh + i, i] = 1.0
    e[0] *= 1.0 / math.sqrt(Dh)
    return jnp.asarray(e).astype(jnp.bfloat16)


def _attention(cs, sorted3):
    r, sb = ATT_R, SUBBLK[cs]
    nb, kb = r // sb, sb + cs
    band, c0 = _band_masks(cs, sb)
    grid = (BH, N // r)
    gg = r // cs
    mspec = pl.BlockSpec((1, r, PK), lambda bh, g: (bh, g, 0))
    halo = pl.BlockSpec((1, cs, PK),
                        lambda bh, g: (bh, jnp.maximum(g * gg - 1, 0), 0))
    out = pl.pallas_call(
        functools.partial(_att_body, cs, r, sb),
        grid=grid,
        in_specs=[mspec, halo,
                  pl.BlockSpec((3, PK, Dh), lambda bh, g: (0, 0, 0)),
                  pl.BlockSpec((1, sb, kb), lambda bh, g: (0, 0, 0)),
                  pl.BlockSpec((1, sb, kb), lambda bh, g: (0, 0, 0))],
        out_specs=pl.BlockSpec((1, r, OW), lambda bh, g: (bh, g, 0)),
        out_shape=jax.ShapeDtypeStruct((BH, N, OW), jnp.float32),
    )(sorted3, sorted3, _esel(), jnp.asarray(band), jnp.asarray(c0))
    return out.reshape(BH * N, OW)


def _sc_unsort_lvl(lvl, ranks_flat, o_l):
    mesh = plsc.VectorSubcoreMesh(core_axis_name="c", subcore_axis_name="s")
    n_per = BH * NCK // NW

    @functools.partial(
        pl.kernel, mesh=mesh,
        out_type=jax.ShapeDtypeStruct((BH * N, OW), jnp.float32),
        scratch_types=[
            pltpu.VMEM((2, CHUNK), jnp.int32),
            pltpu.VMEM((CHUNK, OW), jnp.float32),
            pltpu.VMEM((CHUNK, OW), jnp.float32),
            pltpu.SemaphoreType.DMA,
        ],
        name=f"sc_unsort_l{lvl}",
    )
    def body(ranks_h, o_h, u_h, idx2, rows0, rows1, sem):
        wid = lax.axis_index("s") * NC + lax.axis_index("c")
        rbufs = (rows0, rows1)

        def pair(p, _):
            for j in range(2):
                i = p * 2 + j
                t = wid * n_per + i
                bh = t // NCK
                ck = t - bh * NCK
                n0 = ck * CHUNK
                roff = (bh * NL + lvl) * N + n0
                pltpu.sync_copy(ranks_h.at[pl.ds(roff, CHUNK)],
                                idx2.at[j])
                pltpu.async_copy(o_h.at[idx2.at[j]], rbufs[j], sem).wait()
                pltpu.sync_copy(rbufs[j],
                                u_h.at[pl.ds(bh * N + n0, CHUNK)])
            return 0

        lax.fori_loop(0, n_per // 2, pair, 0)

    return body(ranks_flat, o_l)


def _out_body(u0_ref, u1_ref, u2_ref, woT_ref, bo_ref, o_ref):
    parts = []
    for h in range(H):
        s = (u0_ref[0, h, :, :Dh] + u1_ref[0, h, :, :Dh]
             + u2_ref[0, h, :, :Dh])
        parts.append(s)
    m2 = jnp.concatenate(parts, axis=1).astype(jnp.bfloat16)
    o_ref[0] = (
        jnp.dot(m2, woT_ref[...].astype(jnp.bfloat16),
                preferred_element_type=jnp.float32) + bo_ref[...])


def _oproj(u0, u1, u2, woT3, bo):
    rb = 512
    uspec = pl.BlockSpec((1, H, rb, OW), lambda b, i: (b, 0, i, 0))
    out = pl.pallas_call(
        _out_body,
        grid=(B, N // rb),
        in_specs=[uspec, uspec, uspec,
                  pl.BlockSpec((E, E), lambda b, i: (0, 0)),
                  pl.BlockSpec((1, E), lambda b, i: (0, 0))],
        out_specs=pl.BlockSpec((1, rb, E), lambda b, i: (b, i, 0)),
        out_shape=jax.ShapeDtypeStruct((B, N, E), jnp.float32),
    )(u0.reshape(B, H, N, OW), u1.reshape(B, H, N, OW),
      u2.reshape(B, H, N, OW), woT3, bo.reshape(1, E))
    return out


def kernel(x, Wq, bq, Wk, bk, Wv, bv, Wo, bo, probes):
    bqkv = jnp.stack([bq, bk, bv], axis=0)
    pk4, q4 = _qkv(x, Wq.T, Wk.T, Wv.T, bqkv)

    m3np = np.zeros((LW, LW), np.float32)
    su3np = np.zeros((LW, LW), np.float32)
    for lvl, cs in enumerate(BUCKET_SIZES):
        s = slice(lvl * P, (lvl + 1) * P)
        m3np[s, s] = _class_matrix(max(1, N // cs))
        su3np[s, s] = np.triu(np.ones((P, P), np.float32), 1)
    m3 = jnp.asarray(m3np).astype(jnp.bfloat16)
    su3f = jnp.asarray(su3np)
    su3b = jnp.asarray(su3np).astype(jnp.bfloat16)
    tril = jnp.asarray(np.tril(np.ones((CSB, CSB), np.float32))
                       ).astype(jnp.bfloat16)
    sel3np = np.zeros((NL, LW), np.float32)
    for lvl in range(NL):
        sel3np[lvl, lvl * P:(lvl + 1) * P] = 1.0
    ranks = _ranks(q4, probes, m3, tril, su3f, su3b, jnp.asarray(sel3np))
    ranks_flat = ranks.reshape(-1)

    pk2 = pk4.reshape(BH * N, PK)
    sorted_lvls = [_sc_sort_lvl(lvl, ranks_flat, pk2) for lvl in range(NL)]
    outs = [_attention(cs, sorted_lvls[lvl].reshape(BH, N, PK))
            for lvl, cs in enumerate(BUCKET_SIZES)]
    us = [_sc_unsort_lvl(lvl, ranks_flat, outs[lvl]) for lvl in range(NL)]
    return _oproj(us[0], us[1], us[2], Wo.T * (1.0 / NL), bo)
